# Initial kernel scaffold; baseline (speedup 1.0000x reference)
import jax
import jax.numpy as jnp
from jax.experimental import pallas as pl


def kernel(features, coords, cur_stride, kernel):
    raise NotImplementedError("write your pallas kernel here")


# speedup vs baseline: 2.1613x; 2.1613x over previous
import functools

import jax
import jax.numpy as jnp
from jax import lax
from jax.experimental import pallas as pl
from jax.experimental.pallas import tpu as pltpu
from jax.experimental.pallas import tpu_sc as plsc

N_IN = 200000
C = 64
HALF = 32
NZ = 32
GRID_C = 32
NSEG = GRID_C ** 3

BLK = 512
N_PAD = 204800

NC = 2
NS = 16
PKT = 128
PTS_PER_TILE = N_PAD // NS
NPKT = PTS_PER_TILE // PKT
ROWS_PER_TILE = NSEG // NS
OUT_CHUNK = 256


def _tc_body(stride_ref, coords_ref, feat_ref, kern_ref, out_ref, flat_ref):
    f = feat_ref[...]
    kid = coords_ref[:, 1:2] // stride_ref[0]
    acc = jnp.zeros((BLK, C), jnp.float32)
    for g in range(8):
        parts = [f * (kid == (4 * g + j)).astype(jnp.float32) for j in range(4)]
        fp = jnp.concatenate(parts, axis=1)
        acc = acc + jnp.dot(fp, kern_ref[g], preferred_element_type=jnp.float32)
    out_ref[0] = acc[:, :HALF]
    out_ref[1] = acc[:, HALF:]
    flat_ref[...] = (coords_ref[:, 0:1] * (GRID_C * GRID_C)
                     + coords_ref[:, 2:3] * GRID_C
                     + coords_ref[:, 3:4])


def _tc_stage(stride, coords_p, feats_p, kern_packed):
    return pl.pallas_call(
        _tc_body,
        grid=(N_PAD // BLK,),
        in_specs=[
            pl.BlockSpec(memory_space=pltpu.SMEM),
            pl.BlockSpec((BLK, 4), lambda i: (i, 0)),
            pl.BlockSpec((BLK, C), lambda i: (i, 0)),
            pl.BlockSpec((8, 4 * C, C), lambda i: (0, 0, 0)),
        ],
        out_specs=[
            pl.BlockSpec((2, BLK, HALF), lambda i: (0, i, 0)),
            pl.BlockSpec((BLK, 1), lambda i: (i, 0)),
        ],
        out_shape=[
            jax.ShapeDtypeStruct((2, N_PAD, HALF), jnp.float32),
            jax.ShapeDtypeStruct((N_PAD, 1), jnp.int32),
        ],
    )(stride, coords_p, feats_p, kern_packed)


@functools.lru_cache(maxsize=None)
def _sc_scatter_kernel():
    @functools.partial(
        pl.kernel,
        mesh=plsc.VectorSubcoreMesh(core_axis_name="c", subcore_axis_name="s"),
        out_type=jax.ShapeDtypeStruct((2, NSEG, HALF), jnp.float32),
        scratch_types=[
            pltpu.VMEM((PKT,), jnp.int32),
            pltpu.VMEM((PKT, HALF), jnp.float32),
            pltpu.VMEM((OUT_CHUNK, HALF), jnp.float32),
            pltpu.VMEM_SHARED((NSEG, HALF), jnp.float32),
        ],
        compiler_params=pltpu.CompilerParams(use_tc_tiling_on_sc=False),
    )
    def _sc_scatter(rows_hbm, flat_hbm, zeros_hbm, out_hbm,
                    idx_v, rows_v, buf_v, acc_sh):
        cid = lax.axis_index("c")
        sid = lax.axis_index("s")

        @pl.loop(0, ROWS_PER_TILE // OUT_CHUNK)
        def _init(t):
            sl_t = pl.ds(sid * ROWS_PER_TILE + t * OUT_CHUNK, OUT_CHUNK)
            pltpu.sync_copy(zeros_hbm.at[sl_t], buf_v)
            pltpu.sync_copy(buf_v, acc_sh.at[sl_t])

        plsc.subcore_barrier()

        def run(core):
            @pl.loop(0, NPKT)
            def _packets(j):
                pbase = sid * PTS_PER_TILE + j * PKT
                pltpu.sync_copy(flat_hbm.at[pl.ds(pbase, PKT)], idx_v)
                pltpu.sync_copy(rows_hbm.at[core, pl.ds(pbase, PKT)], rows_v)
                pltpu.sync_copy(rows_v, acc_sh.at[idx_v], add=True)

        @pl.when(cid == 0)
        def _():
            run(0)

        @pl.when(cid == 1)
        def _():
            run(1)

        plsc.subcore_barrier()

        @pl.loop(0, ROWS_PER_TILE // OUT_CHUNK)
        def _out(t):
            sl_t = pl.ds(sid * ROWS_PER_TILE + t * OUT_CHUNK, OUT_CHUNK)
            pltpu.sync_copy(acc_sh.at[sl_t], buf_v)

            @pl.when(cid == 0)
            def _():
                pltpu.sync_copy(buf_v, out_hbm.at[0, sl_t])

            @pl.when(cid == 1)
            def _():
                pltpu.sync_copy(buf_v, out_hbm.at[1, sl_t])

    return _sc_scatter


def kernel(features, coords, cur_stride, kernel):
    n = features.shape[0]
    feats_p = jnp.pad(features, ((0, N_PAD - n), (0, 0)))
    coords_p = jnp.pad(coords, ((0, N_PAD - n), (0, 0)))
    stride = jnp.asarray(cur_stride, jnp.int32).reshape(1)
    kern_packed = kernel.reshape(8, 4 * C, C)

    rows2, flat = _tc_stage(stride, coords_p, feats_p, kern_packed)
    zeros = jnp.zeros((NSEG, HALF), jnp.float32)
    bev2 = _sc_scatter_kernel()(rows2, flat.reshape(N_PAD), zeros)
    return bev2.transpose(1, 0, 2).reshape(NSEG, C)


# speedup vs baseline: 5.2029x; 2.4073x over previous
import functools

import jax
import jax.numpy as jnp
from jax import lax
from jax.experimental import pallas as pl
from jax.experimental.pallas import tpu as pltpu
from jax.experimental.pallas import tpu_sc as plsc

N_IN = 200000
C = 64
HALF = 32
NZ = 32
GRID_C = 32
NSEG = GRID_C ** 3

BLK = 512
N_PAD = 204800

NC = 2
NS = 16
PKT = 128
PTS_PER_TILE = N_PAD // NS
NPKT = PTS_PER_TILE // PKT
ROWS_PER_TILE = NSEG // NS
OUT_CHUNK = 256


def _tc_body(coords_ref, feat_ref, kern_ref, out_ref, flat_ref):
    f = feat_ref[...]
    kid = coords_ref[:, 1:2]
    f4 = jnp.concatenate([f, f, f, f], axis=1).astype(jnp.bfloat16)
    kidb = jnp.broadcast_to(kid, (BLK, 4 * C)).astype(jnp.bfloat16)
    zpat = (lax.broadcasted_iota(jnp.int32, (BLK, 4 * C), 1) // C
            ).astype(jnp.bfloat16)
    d = kidb - zpat
    acc = jnp.zeros((BLK, C), jnp.float32)
    for g in range(8):
        fp = jnp.where(d == 4 * g, f4, jnp.bfloat16(0))
        acc = acc + jnp.dot(fp, kern_ref[g], preferred_element_type=jnp.float32)
    out_ref[0] = acc[:, :HALF]
    out_ref[1] = acc[:, HALF:]
    flat_ref[...] = (coords_ref[:, 0:1] * (GRID_C * GRID_C)
                     + coords_ref[:, 2:3] * GRID_C
                     + coords_ref[:, 3:4])


def _tc_stage(coords_p, feats_p, kern_packed):
    return pl.pallas_call(
        _tc_body,
        grid=(N_PAD // BLK,),
        in_specs=[
            pl.BlockSpec((BLK, 4), lambda i: (i, 0)),
            pl.BlockSpec((BLK, C), lambda i: (i, 0)),
            pl.BlockSpec((8, 4 * C, C), lambda i: (0, 0, 0)),
        ],
        out_specs=[
            pl.BlockSpec((2, BLK, HALF), lambda i: (0, i, 0)),
            pl.BlockSpec((BLK, 1), lambda i: (i, 0)),
        ],
        out_shape=[
            jax.ShapeDtypeStruct((2, N_PAD, HALF), jnp.float32),
            jax.ShapeDtypeStruct((N_PAD, 1), jnp.int32),
        ],
    )(coords_p, feats_p, kern_packed)


@functools.lru_cache(maxsize=None)
def _sc_scatter_kernel():
    @functools.partial(
        pl.kernel,
        mesh=plsc.VectorSubcoreMesh(core_axis_name="c", subcore_axis_name="s"),
        out_type=jax.ShapeDtypeStruct((2, NSEG, HALF), jnp.float32),
        scratch_types=[
            pltpu.VMEM((PKT,), jnp.int32),
            pltpu.VMEM((PKT, HALF), jnp.float32),
            pltpu.VMEM((OUT_CHUNK, HALF), jnp.float32),
            pltpu.VMEM_SHARED((NSEG, HALF), jnp.float32),
        ],
        compiler_params=pltpu.CompilerParams(use_tc_tiling_on_sc=False),
    )
    def _sc_scatter(rows_hbm, flat_hbm, zeros_hbm, out_hbm,
                    idx_v, rows_v, buf_v, acc_sh):
        cid = lax.axis_index("c")
        sid = lax.axis_index("s")

        @pl.loop(0, ROWS_PER_TILE // OUT_CHUNK)
        def _init(t):
            sl_t = pl.ds(sid * ROWS_PER_TILE + t * OUT_CHUNK, OUT_CHUNK)
            pltpu.sync_copy(zeros_hbm.at[sl_t], buf_v)
            pltpu.sync_copy(buf_v, acc_sh.at[sl_t])

        plsc.subcore_barrier()

        def run(core):
            @pl.loop(0, NPKT)
            def _packets(j):
                pbase = sid * PTS_PER_TILE + j * PKT
                pltpu.sync_copy(flat_hbm.at[pl.ds(pbase, PKT)], idx_v)
                pltpu.sync_copy(rows_hbm.at[core, pl.ds(pbase, PKT)], rows_v)
                pltpu.sync_copy(rows_v, acc_sh.at[idx_v], add=True)

        @pl.when(cid == 0)
        def _():
            run(0)

        @pl.when(cid == 1)
        def _():
            run(1)

        plsc.subcore_barrier()

        @pl.loop(0, ROWS_PER_TILE // OUT_CHUNK)
        def _out(t):
            sl_t = pl.ds(sid * ROWS_PER_TILE + t * OUT_CHUNK, OUT_CHUNK)
            pltpu.sync_copy(acc_sh.at[sl_t], buf_v)

            @pl.when(cid == 0)
            def _():
                pltpu.sync_copy(buf_v, out_hbm.at[0, sl_t])

            @pl.when(cid == 1)
            def _():
                pltpu.sync_copy(buf_v, out_hbm.at[1, sl_t])

    return _sc_scatter


def kernel(features, coords, cur_stride, kernel):
    n = features.shape[0]
    feats_p = jnp.pad(features, ((0, N_PAD - n), (0, 0)))
    coords_p = jnp.pad(coords, ((0, N_PAD - n), (0, 0)))
    zsel = jnp.arange(NZ, dtype=jnp.int32) // jnp.asarray(cur_stride, jnp.int32)
    kern_eff = jnp.take(kernel, zsel, axis=0)
    kern_packed = kern_eff.reshape(8, 4 * C, C).astype(jnp.bfloat16)

    rows2, flat = _tc_stage(coords_p, feats_p, kern_packed)
    zeros = jnp.zeros((NSEG, HALF), jnp.float32)
    bev2 = _sc_scatter_kernel()(rows2, flat.reshape(N_PAD), zeros)
    return bev2.transpose(1, 0, 2).reshape(NSEG, C)


# speedup vs baseline: 5.3017x; 1.0190x over previous
import functools

import jax
import jax.numpy as jnp
from jax import lax
from jax.experimental import pallas as pl
from jax.experimental.pallas import tpu as pltpu
from jax.experimental.pallas import tpu_sc as plsc

N_IN = 200000
C = 64
HALF = 32
NZ = 32
GRID_C = 32
NSEG = GRID_C ** 3

BLK = 512
N_PAD = 204800

NC = 2
NS = 16
PKT = 128
PTS_PER_TILE = N_PAD // NS
NPKT = PTS_PER_TILE // PKT
ROWS_PER_TILE = NSEG // NS
OUT_CHUNK = 256


def _tc_body(coords_ref, feat_ref, kern_ref, out_ref, flat_ref):
    f = feat_ref[...]
    kid = coords_ref[:, 1:2]
    f4 = jnp.concatenate([f, f, f, f], axis=1).astype(jnp.bfloat16)
    kidb = jnp.broadcast_to(kid, (BLK, 4 * C)).astype(jnp.bfloat16)
    zpat = (lax.broadcasted_iota(jnp.int32, (BLK, 4 * C), 1) // C
            ).astype(jnp.bfloat16)
    d = kidb - zpat
    acc = jnp.zeros((BLK, C), jnp.float32)
    for g in range(8):
        fp = jnp.where(d == 4 * g, f4, jnp.bfloat16(0))
        acc = acc + jnp.dot(fp, kern_ref[g], preferred_element_type=jnp.float32)
    out_ref[0] = acc[:, :HALF]
    out_ref[1] = acc[:, HALF:]
    flat_ref[...] = (coords_ref[:, 0:1] * (GRID_C * GRID_C)
                     + coords_ref[:, 2:3] * GRID_C
                     + coords_ref[:, 3:4])


def _tc_stage(coords_p, feats_p, kern_packed):
    return pl.pallas_call(
        _tc_body,
        grid=(N_PAD // BLK,),
        in_specs=[
            pl.BlockSpec((BLK, 4), lambda i: (i, 0)),
            pl.BlockSpec((BLK, C), lambda i: (i, 0)),
            pl.BlockSpec((8, 4 * C, C), lambda i: (0, 0, 0)),
        ],
        out_specs=[
            pl.BlockSpec((2, BLK, HALF), lambda i: (0, i, 0)),
            pl.BlockSpec((BLK, 1), lambda i: (i, 0)),
        ],
        out_shape=[
            jax.ShapeDtypeStruct((2, N_PAD, HALF), jnp.float32),
            jax.ShapeDtypeStruct((N_PAD, 1), jnp.int32),
        ],
    )(coords_p, feats_p, kern_packed)


@functools.lru_cache(maxsize=None)
def _sc_scatter_kernel():
    @functools.partial(
        pl.kernel,
        mesh=plsc.VectorSubcoreMesh(core_axis_name="c", subcore_axis_name="s"),
        out_type=jax.ShapeDtypeStruct((NSEG, C), jnp.float32),
        scratch_types=[
            pltpu.VMEM((PKT,), jnp.int32),
            pltpu.VMEM((PKT, HALF), jnp.float32),
            pltpu.VMEM((OUT_CHUNK, HALF), jnp.float32),
            pltpu.VMEM_SHARED((NSEG, HALF), jnp.float32),
        ],
        compiler_params=pltpu.CompilerParams(use_tc_tiling_on_sc=False),
    )
    def _sc_scatter(rows_hbm, flat_hbm, out_hbm,
                    idx_v, rows_v, buf_v, acc_sh):
        cid = lax.axis_index("c")
        sid = lax.axis_index("s")

        zv = jnp.zeros((16,), jnp.float32)

        @pl.loop(0, OUT_CHUNK)
        def _zrow(r):
            buf_v[r, pl.ds(0, 16)] = zv
            buf_v[r, pl.ds(16, 16)] = zv

        @pl.loop(0, ROWS_PER_TILE // OUT_CHUNK)
        def _init(t):
            sl_t = pl.ds(sid * ROWS_PER_TILE + t * OUT_CHUNK, OUT_CHUNK)
            pltpu.sync_copy(buf_v, acc_sh.at[sl_t])

        plsc.subcore_barrier()

        def run(core):
            @pl.loop(0, NPKT)
            def _packets(j):
                pbase = sid * PTS_PER_TILE + j * PKT
                pltpu.sync_copy(flat_hbm.at[pl.ds(pbase, PKT)], idx_v)
                pltpu.sync_copy(rows_hbm.at[core, pl.ds(pbase, PKT)], rows_v)
                pltpu.sync_copy(rows_v, acc_sh.at[idx_v], add=True)

        @pl.when(cid == 0)
        def _():
            run(0)

        @pl.when(cid == 1)
        def _():
            run(1)

        plsc.subcore_barrier()

        @pl.loop(0, ROWS_PER_TILE // OUT_CHUNK)
        def _out(t):
            sl_t = pl.ds(sid * ROWS_PER_TILE + t * OUT_CHUNK, OUT_CHUNK)
            pltpu.sync_copy(acc_sh.at[sl_t], buf_v)

            @pl.when(cid == 0)
            def _():
                pltpu.sync_copy(buf_v, out_hbm.at[sl_t, pl.ds(0, HALF)])

            @pl.when(cid == 1)
            def _():
                pltpu.sync_copy(buf_v, out_hbm.at[sl_t, pl.ds(HALF, HALF)])

    return _sc_scatter


def kernel(features, coords, cur_stride, kernel):
    n = features.shape[0]
    feats_p = jnp.pad(features, ((0, N_PAD - n), (0, 0)))
    coords_p = jnp.pad(coords, ((0, N_PAD - n), (0, 0)))
    zsel = jnp.arange(NZ, dtype=jnp.int32) // jnp.asarray(cur_stride, jnp.int32)
    kern_eff = jnp.take(kernel, zsel, axis=0)
    kern_packed = kern_eff.reshape(8, 4 * C, C).astype(jnp.bfloat16)

    rows2, flat = _tc_stage(coords_p, feats_p, kern_packed)
    return _sc_scatter_kernel()(rows2, flat.reshape(N_PAD))


# speedup vs baseline: 5.9129x; 1.1153x over previous
import functools

import jax
import jax.numpy as jnp
from jax import lax
from jax.experimental import pallas as pl
from jax.experimental.pallas import tpu as pltpu
from jax.experimental.pallas import tpu_sc as plsc

N_IN = 200000
C = 64
HALF = 32
NZ = 32
GRID_C = 32
NSEG = GRID_C ** 3

BLK = 512
N_PAD = 200704

NC = 2
NS = 16
PKT = 128
PTS_PER_TILE = N_PAD // NS
NPKT = PTS_PER_TILE // PKT
ROWS_PER_TILE = NSEG // NS
OUT_CHUNK = 256


def _tc_body(coords_ref, feat_ref, kern_ref, out_ref, flat_ref):
    f = feat_ref[...]
    kid = coords_ref[:, 1:2]
    f4 = jnp.concatenate([f, f, f, f], axis=1).astype(jnp.bfloat16)
    kidb = jnp.broadcast_to(kid, (BLK, 4 * C)).astype(jnp.bfloat16)
    zpat = (lax.broadcasted_iota(jnp.int32, (BLK, 4 * C), 1) // C
            ).astype(jnp.bfloat16)
    d = kidb - zpat
    acc = jnp.zeros((BLK, C), jnp.float32)
    for g in range(8):
        fp = jnp.where(d == 4 * g, f4, jnp.bfloat16(0))
        acc = acc + jnp.dot(fp, kern_ref[g], preferred_element_type=jnp.float32)
    rid = (lax.broadcasted_iota(jnp.int32, (BLK, 1), 0)
           + pl.program_id(0) * BLK)
    valid = rid < N_IN
    out_ref[...] = jnp.where(jnp.broadcast_to(valid, (BLK, C)), acc, 0.0)
    flat = (coords_ref[:, 0:1] * (GRID_C * GRID_C)
            + coords_ref[:, 2:3] * GRID_C
            + coords_ref[:, 3:4])
    flat_ref[...] = jnp.where(valid, flat, 0)


def _tc_stage(coords_p, feats_p, kern_packed):
    return pl.pallas_call(
        _tc_body,
        grid=(N_PAD // BLK,),
        in_specs=[
            pl.BlockSpec((BLK, 4), lambda i: (i, 0)),
            pl.BlockSpec((BLK, C), lambda i: (i, 0)),
            pl.BlockSpec((8, 4 * C, C), lambda i: (0, 0, 0)),
        ],
        out_specs=[
            pl.BlockSpec((BLK, C), lambda i: (i, 0)),
            pl.BlockSpec((BLK, 1), lambda i: (i, 0)),
        ],
        out_shape=[
            jax.ShapeDtypeStruct((N_PAD, C), jnp.float32),
            jax.ShapeDtypeStruct((N_PAD, 1), jnp.int32),
        ],
    )(coords_p, feats_p, kern_packed)


@functools.lru_cache(maxsize=None)
def _sc_scatter_kernel():
    @functools.partial(
        pl.kernel,
        mesh=plsc.VectorSubcoreMesh(core_axis_name="c", subcore_axis_name="s"),
        out_type=jax.ShapeDtypeStruct((NSEG, C), jnp.float32),
        scratch_types=[
            pltpu.VMEM((PKT,), jnp.int32),
            pltpu.VMEM((PKT, HALF), jnp.float32),
            pltpu.VMEM((OUT_CHUNK, HALF), jnp.float32),
            pltpu.VMEM_SHARED((NSEG, HALF), jnp.float32),
        ],
        compiler_params=pltpu.CompilerParams(use_tc_tiling_on_sc=False),
    )
    def _sc_scatter(rows_hbm, flat_hbm, out_hbm,
                    idx_v, rows_v, buf_v, acc_sh):
        cid = lax.axis_index("c")
        sid = lax.axis_index("s")

        zv = jnp.zeros((16,), jnp.float32)

        @pl.loop(0, OUT_CHUNK)
        def _zrow(r):
            buf_v[r, pl.ds(0, 16)] = zv
            buf_v[r, pl.ds(16, 16)] = zv

        @pl.loop(0, ROWS_PER_TILE // OUT_CHUNK)
        def _init(t):
            sl_t = pl.ds(sid * ROWS_PER_TILE + t * OUT_CHUNK, OUT_CHUNK)
            pltpu.sync_copy(buf_v, acc_sh.at[sl_t])

        plsc.subcore_barrier()

        def run(coff):
            @pl.loop(0, NPKT)
            def _packets(j):
                pbase = sid * PTS_PER_TILE + j * PKT
                pltpu.sync_copy(flat_hbm.at[pl.ds(pbase, PKT)], idx_v)
                pltpu.sync_copy(
                    rows_hbm.at[pl.ds(pbase, PKT), pl.ds(coff, HALF)], rows_v)
                pltpu.sync_copy(rows_v, acc_sh.at[idx_v], add=True)

        @pl.when(cid == 0)
        def _():
            run(0)

        @pl.when(cid == 1)
        def _():
            run(HALF)

        plsc.subcore_barrier()

        @pl.loop(0, ROWS_PER_TILE // OUT_CHUNK)
        def _out(t):
            sl_t = pl.ds(sid * ROWS_PER_TILE + t * OUT_CHUNK, OUT_CHUNK)
            pltpu.sync_copy(acc_sh.at[sl_t], buf_v)

            @pl.when(cid == 0)
            def _():
                pltpu.sync_copy(buf_v, out_hbm.at[sl_t, pl.ds(0, HALF)])

            @pl.when(cid == 1)
            def _():
                pltpu.sync_copy(buf_v, out_hbm.at[sl_t, pl.ds(HALF, HALF)])

    return _sc_scatter


def kernel(features, coords, cur_stride, kernel):
    zsel = jnp.arange(NZ, dtype=jnp.int32) // jnp.asarray(cur_stride, jnp.int32)
    kern_eff = jnp.take(kernel, zsel, axis=0)
    kern_packed = kern_eff.reshape(8, 4 * C, C).astype(jnp.bfloat16)

    n = features.shape[0]
    feats_p = jnp.pad(features, ((0, N_PAD - n), (0, 0)))
    coords_p = jnp.pad(coords, ((0, N_PAD - n), (0, 0)))
    rows, flat = _tc_stage(coords_p, feats_p, kern_packed)
    return _sc_scatter_kernel()(rows, flat.reshape(N_PAD))


# speedup vs baseline: 6.6975x; 1.1327x over previous
import functools

import jax
import jax.numpy as jnp
from jax import lax
from jax.experimental import pallas as pl
from jax.experimental.pallas import tpu as pltpu
from jax.experimental.pallas import tpu_sc as plsc

N_IN = 200000
C = 64
HALF = 32
NZ = 32
GRID_C = 32
NSEG = GRID_C ** 3

BLK = 1024
N_PAD = 200704

NC = 2
NS = 16
PKT = 128
PTS_PER_TILE = N_PAD // NS
NPKT = PTS_PER_TILE // PKT
ROWS_PER_TILE = NSEG // NS
OUT_CHUNK = 256


def _tc_body(coords_ref, feat_ref, kern_ref, out_ref, flat_ref):
    f = feat_ref[...]
    kid = coords_ref[:, 1:2]
    f4 = jnp.concatenate([f, f, f, f], axis=1).astype(jnp.bfloat16)
    kidb = jnp.broadcast_to(kid, (BLK, 4 * C)).astype(jnp.bfloat16)
    zpat = (lax.broadcasted_iota(jnp.int32, (BLK, 4 * C), 1) // C
            ).astype(jnp.bfloat16)
    d = kidb - zpat
    acc = jnp.zeros((BLK, C), jnp.float32)
    for g in range(8):
        fp = jnp.where(d == 4 * g, f4, jnp.bfloat16(0))
        acc = acc + jnp.dot(fp, kern_ref[g], preferred_element_type=jnp.float32)
    rid = (lax.broadcasted_iota(jnp.int32, (BLK, 1), 0)
           + pl.program_id(0) * BLK)
    valid = rid < N_IN
    out_ref[...] = jnp.where(jnp.broadcast_to(valid, (BLK, C)), acc, 0.0)
    flat = (coords_ref[:, 0:1] * (GRID_C * GRID_C)
            + coords_ref[:, 2:3] * GRID_C
            + coords_ref[:, 3:4])
    flat_ref[...] = jnp.where(valid, flat, 0)


def _tc_stage(coords_p, feats_p, kern_packed):
    return pl.pallas_call(
        _tc_body,
        grid=(N_PAD // BLK,),
        in_specs=[
            pl.BlockSpec((BLK, 4), lambda i: (i, 0)),
            pl.BlockSpec((BLK, C), lambda i: (i, 0)),
            pl.BlockSpec((8, 4 * C, C), lambda i: (0, 0, 0)),
        ],
        out_specs=[
            pl.BlockSpec((BLK, C), lambda i: (i, 0)),
            pl.BlockSpec((BLK, 1), lambda i: (i, 0)),
        ],
        out_shape=[
            jax.ShapeDtypeStruct((N_PAD, C), jnp.float32),
            jax.ShapeDtypeStruct((N_PAD, 1), jnp.int32),
        ],
    )(coords_p, feats_p, kern_packed)


@functools.lru_cache(maxsize=None)
def _sc_scatter_kernel():
    @functools.partial(
        pl.kernel,
        mesh=plsc.VectorSubcoreMesh(core_axis_name="c", subcore_axis_name="s"),
        out_type=jax.ShapeDtypeStruct((NSEG, C), jnp.float32),
        scratch_types=[
            pltpu.VMEM((PKT,), jnp.int32),
            pltpu.VMEM((PKT, HALF), jnp.float32),
            pltpu.VMEM((OUT_CHUNK, HALF), jnp.float32),
            pltpu.VMEM_SHARED((NSEG, HALF), jnp.float32),
        ],
        compiler_params=pltpu.CompilerParams(use_tc_tiling_on_sc=False),
    )
    def _sc_scatter(rows_hbm, flat_hbm, out_hbm,
                    idx_v, rows_v, buf_v, acc_sh):
        cid = lax.axis_index("c")
        sid = lax.axis_index("s")

        zv = jnp.zeros((16,), jnp.float32)

        @pl.loop(0, OUT_CHUNK)
        def _zrow(r):
            buf_v[r, pl.ds(0, 16)] = zv
            buf_v[r, pl.ds(16, 16)] = zv

        @pl.loop(0, ROWS_PER_TILE // OUT_CHUNK)
        def _init(t):
            sl_t = pl.ds(sid * ROWS_PER_TILE + t * OUT_CHUNK, OUT_CHUNK)
            pltpu.sync_copy(buf_v, acc_sh.at[sl_t])

        plsc.subcore_barrier()

        def run(coff):
            @pl.loop(0, NPKT)
            def _packets(j):
                pbase = sid * PTS_PER_TILE + j * PKT
                pltpu.sync_copy(flat_hbm.at[pl.ds(pbase, PKT)], idx_v)
                pltpu.sync_copy(
                    rows_hbm.at[pl.ds(pbase, PKT), pl.ds(coff, HALF)], rows_v)
                pltpu.sync_copy(rows_v, acc_sh.at[idx_v], add=True)

        @pl.when(cid == 0)
        def _():
            run(0)

        @pl.when(cid == 1)
        def _():
            run(HALF)

        plsc.subcore_barrier()

        @pl.loop(0, ROWS_PER_TILE // OUT_CHUNK)
        def _out(t):
            sl_t = pl.ds(sid * ROWS_PER_TILE + t * OUT_CHUNK, OUT_CHUNK)
            pltpu.sync_copy(acc_sh.at[sl_t], buf_v)

            @pl.when(cid == 0)
            def _():
                pltpu.sync_copy(buf_v, out_hbm.at[sl_t, pl.ds(0, HALF)])

            @pl.when(cid == 1)
            def _():
                pltpu.sync_copy(buf_v, out_hbm.at[sl_t, pl.ds(HALF, HALF)])

    return _sc_scatter


def kernel(features, coords, cur_stride, kernel):
    zsel = jnp.arange(NZ, dtype=jnp.int32) // jnp.asarray(cur_stride, jnp.int32)
    kern_eff = jnp.take(kernel, zsel, axis=0)
    kern_packed = kern_eff.reshape(8, 4 * C, C).astype(jnp.bfloat16)

    n = features.shape[0]
    feats_p = jnp.pad(features, ((0, N_PAD - n), (0, 0)))
    coords_p = jnp.pad(coords, ((0, N_PAD - n), (0, 0)))
    rows, flat = _tc_stage(coords_p, feats_p, kern_packed)
    return _sc_scatter_kernel()(rows, flat.reshape(N_PAD))


# speedup vs baseline: 7.0451x; 1.0519x over previous
import functools

import jax
import jax.numpy as jnp
from jax import lax
from jax.experimental import pallas as pl
from jax.experimental.pallas import tpu as pltpu
from jax.experimental.pallas import tpu_sc as plsc

N_IN = 200000
C = 64
HALF = 32
NZ = 32
GRID_C = 32
NSEG = GRID_C ** 3

BLK = 2048
N_PAD = 200704

NC = 2
NS = 16
PKT = 128
PTS_PER_TILE = N_PAD // NS
NPKT = PTS_PER_TILE // PKT
ROWS_PER_TILE = NSEG // NS
OUT_CHUNK = 256


def _tc_body(coords_ref, feat_ref, kern_ref, out_ref, flat_ref):
    f = feat_ref[...]
    kid = coords_ref[:, 1:2]
    f4 = jnp.concatenate([f, f, f, f], axis=1).astype(jnp.bfloat16)
    kidb = jnp.broadcast_to(kid, (BLK, 4 * C)).astype(jnp.bfloat16)
    zpat = (lax.broadcasted_iota(jnp.int32, (BLK, 4 * C), 1) // C
            ).astype(jnp.bfloat16)
    d = kidb - zpat
    acc = jnp.zeros((BLK, C), jnp.float32)
    for g in range(8):
        fp = jnp.where(d == 4 * g, f4, jnp.bfloat16(0))
        acc = acc + jnp.dot(fp, kern_ref[g], preferred_element_type=jnp.float32)
    rid = (lax.broadcasted_iota(jnp.int32, (BLK, 1), 0)
           + pl.program_id(0) * BLK)
    valid = rid < N_IN
    out_ref[...] = jnp.where(jnp.broadcast_to(valid, (BLK, C)), acc, 0.0)
    flat = (coords_ref[:, 0:1] * (GRID_C * GRID_C)
            + coords_ref[:, 2:3] * GRID_C
            + coords_ref[:, 3:4])
    flat_ref[...] = jnp.where(valid, flat, 0)


def _tc_stage(coords_p, feats_p, kern_packed):
    return pl.pallas_call(
        _tc_body,
        grid=(N_PAD // BLK,),
        in_specs=[
            pl.BlockSpec((BLK, 4), lambda i: (i, 0)),
            pl.BlockSpec((BLK, C), lambda i: (i, 0)),
            pl.BlockSpec((8, 4 * C, C), lambda i: (0, 0, 0)),
        ],
        out_specs=[
            pl.BlockSpec((BLK, C), lambda i: (i, 0)),
            pl.BlockSpec((BLK, 1), lambda i: (i, 0)),
        ],
        out_shape=[
            jax.ShapeDtypeStruct((N_PAD, C), jnp.float32),
            jax.ShapeDtypeStruct((N_PAD, 1), jnp.int32),
        ],
    )(coords_p, feats_p, kern_packed)


@functools.lru_cache(maxsize=None)
def _sc_scatter_kernel():
    @functools.partial(
        pl.kernel,
        mesh=plsc.VectorSubcoreMesh(core_axis_name="c", subcore_axis_name="s"),
        out_type=jax.ShapeDtypeStruct((NSEG, C), jnp.float32),
        scratch_types=[
            pltpu.VMEM((PKT,), jnp.int32),
            pltpu.VMEM((PKT, HALF), jnp.float32),
            pltpu.VMEM((OUT_CHUNK, HALF), jnp.float32),
            pltpu.VMEM_SHARED((NSEG, HALF), jnp.float32),
        ],
        compiler_params=pltpu.CompilerParams(use_tc_tiling_on_sc=False),
    )
    def _sc_scatter(rows_hbm, flat_hbm, out_hbm,
                    idx_v, rows_v, buf_v, acc_sh):
        cid = lax.axis_index("c")
        sid = lax.axis_index("s")

        zv = jnp.zeros((16,), jnp.float32)

        @pl.loop(0, OUT_CHUNK)
        def _zrow(r):
            buf_v[r, pl.ds(0, 16)] = zv
            buf_v[r, pl.ds(16, 16)] = zv

        @pl.loop(0, ROWS_PER_TILE // OUT_CHUNK)
        def _init(t):
            sl_t = pl.ds(sid * ROWS_PER_TILE + t * OUT_CHUNK, OUT_CHUNK)
            pltpu.sync_copy(buf_v, acc_sh.at[sl_t])

        plsc.subcore_barrier()

        def run(coff):
            @pl.loop(0, NPKT)
            def _packets(j):
                pbase = sid * PTS_PER_TILE + j * PKT
                pltpu.sync_copy(flat_hbm.at[pl.ds(pbase, PKT)], idx_v)
                pltpu.sync_copy(
                    rows_hbm.at[pl.ds(pbase, PKT), pl.ds(coff, HALF)], rows_v)
                pltpu.sync_copy(rows_v, acc_sh.at[idx_v], add=True)

        @pl.when(cid == 0)
        def _():
            run(0)

        @pl.when(cid == 1)
        def _():
            run(HALF)

        plsc.subcore_barrier()

        @pl.loop(0, ROWS_PER_TILE // OUT_CHUNK)
        def _out(t):
            sl_t = pl.ds(sid * ROWS_PER_TILE + t * OUT_CHUNK, OUT_CHUNK)
            pltpu.sync_copy(acc_sh.at[sl_t], buf_v)

            @pl.when(cid == 0)
            def _():
                pltpu.sync_copy(buf_v, out_hbm.at[sl_t, pl.ds(0, HALF)])

            @pl.when(cid == 1)
            def _():
                pltpu.sync_copy(buf_v, out_hbm.at[sl_t, pl.ds(HALF, HALF)])

    return _sc_scatter


def kernel(features, coords, cur_stride, kernel):
    zsel = jnp.arange(NZ, dtype=jnp.int32) // jnp.asarray(cur_stride, jnp.int32)
    kern_eff = jnp.take(kernel, zsel, axis=0)
    kern_packed = kern_eff.reshape(8, 4 * C, C).astype(jnp.bfloat16)

    n = features.shape[0]
    feats_p = jnp.pad(features, ((0, N_PAD - n), (0, 0)))
    coords_p = jnp.pad(coords, ((0, N_PAD - n), (0, 0)))
    rows, flat = _tc_stage(coords_p, feats_p, kern_packed)
    return _sc_scatter_kernel()(rows, flat.reshape(N_PAD))
